# Initial kernel scaffold; baseline (speedup 1.0000x reference)
import jax
import jax.numpy as jnp
from jax.experimental import pallas as pl


def kernel(x, edge_index, W1, b1, W2, b2):
    raise NotImplementedError("write your pallas kernel here")


# speedup vs baseline: 92.7832x; 92.7832x over previous
import functools

import jax
import jax.numpy as jnp
from jax import lax
from jax.experimental import pallas as pl
from jax.experimental.pallas import tpu as pltpu
from jax.experimental.pallas import tpu_sc as plsc

N = 10000
E = 320000
C = 128
H = 4

NTILES = 32
EPT = E // NTILES
GROUPS = EPT // 16

_SC_PARAMS = pltpu.CompilerParams(needs_layout_passes=False)
_MESH = plsc.VectorSubcoreMesh(core_axis_name="c", subcore_axis_name="s")


def _wid():
    return lax.axis_index("s") * 2 + lax.axis_index("c")


@functools.partial(
    pl.kernel,
    out_type=jax.ShapeDtypeStruct((NTILES, N), jnp.float32),
    mesh=_MESH,
    compiler_params=_SC_PARAMS,
    scratch_types=[pltpu.VMEM((EPT,), jnp.int32),
                   pltpu.VMEM((N,), jnp.float32)],
)
def _sc_degree(dst_hbm, zeros_hbm, out_hbm, dst_v, acc_v):
    w = _wid()
    pltpu.sync_copy(zeros_hbm, acc_v)
    pltpu.sync_copy(dst_hbm.at[pl.ds(w * EPT, EPT)], dst_v)
    ones = jnp.ones((16,), jnp.float32)

    def body(i, carry):
        d = dst_v[pl.ds(i * 16, 16)]
        plsc.addupdate_scatter(acc_v, [d], ones)
        return carry

    lax.fori_loop(0, GROUPS, body, 0)
    pltpu.sync_copy(acc_v, out_hbm.at[w])


def _make_sc_agg(F):
    FN = F * N

    @functools.partial(
        pl.kernel,
        out_type=jax.ShapeDtypeStruct((NTILES, FN), jnp.float32),
        mesh=_MESH,
        compiler_params=_SC_PARAMS,
        scratch_types=[pltpu.VMEM((EPT,), jnp.int32),
                       pltpu.VMEM((EPT,), jnp.int32),
                       pltpu.VMEM((FN,), jnp.float32),
                       pltpu.VMEM((FN,), jnp.float32)],
    )
    def agg(q_hbm, src_hbm, dst_hbm, zeros_hbm, out_hbm,
            src_v, dst_v, q_v, acc_v):
        w = _wid()
        pltpu.sync_copy(q_hbm, q_v)
        pltpu.sync_copy(zeros_hbm, acc_v)
        pltpu.sync_copy(src_hbm.at[pl.ds(w * EPT, EPT)], src_v)
        pltpu.sync_copy(dst_hbm.at[pl.ds(w * EPT, EPT)], dst_v)

        def body(i, carry):
            s = src_v[pl.ds(i * 16, 16)]
            d = dst_v[pl.ds(i * 16, 16)]
            for j in range(F):
                si = s if j == 0 else s + (j * N)
                di = d if j == 0 else d + (j * N)
                g = plsc.load_gather(q_v, [si])
                plsc.addupdate_scatter(acc_v, [di], g)
            return carry

        lax.fori_loop(0, GROUPS, body, 0)
        pltpu.sync_copy(acc_v, out_hbm.at[w])

    return agg


_sc_agg4 = _make_sc_agg(H)
_sc_agg1 = _make_sc_agg(1)


def _tc1_body(degp_ref, x_ref, w1_ref, q1_ref, dinv_ref):
    deg = jnp.sum(degp_ref[...], axis=0, keepdims=True) + 1.0
    dinv = lax.rsqrt(deg)
    xwt = lax.dot_general(w1_ref[...], x_ref[...],
                          (((1,), (1,)), ((), ())),
                          preferred_element_type=jnp.float32)
    q1_ref[...] = xwt * dinv
    dinv_ref[...] = dinv


_tc1 = pl.pallas_call(
    _tc1_body,
    out_shape=(jax.ShapeDtypeStruct((H, N), jnp.float32),
               jax.ShapeDtypeStruct((1, N), jnp.float32)))


def _tc2_body(accp_ref, q1_ref, dinv_ref, b1_ref, w2_ref, q2_ref):
    acc = jnp.sum(accp_ref[...], axis=0) + q1_ref[...]
    dinv = dinv_ref[...]
    h = jnp.maximum(acc * dinv + b1_ref[...], 0.0)
    hwt = lax.dot_general(w2_ref[...], h, (((1,), (0,)), ((), ())),
                          preferred_element_type=jnp.float32)
    q2_ref[...] = hwt * dinv


_tc2 = pl.pallas_call(
    _tc2_body,
    out_shape=jax.ShapeDtypeStruct((1, N), jnp.float32))


def _tc3_body(accp_ref, q2_ref, dinv_ref, b2_ref, out_ref):
    acc = jnp.sum(accp_ref[...], axis=0, keepdims=True) + q2_ref[...]
    z = acc * dinv_ref[...] + b2_ref[...]
    out_ref[...] = 1.0 / (1.0 + jnp.exp(-z))


_tc3 = pl.pallas_call(
    _tc3_body,
    out_shape=jax.ShapeDtypeStruct((1, N), jnp.float32))


def kernel(x, edge_index, W1, b1, W2, b2):
    src = edge_index[0].astype(jnp.int32)
    dst = edge_index[1].astype(jnp.int32)
    zn = jnp.zeros((N,), jnp.float32)
    zhn = jnp.zeros((H * N,), jnp.float32)

    degp = _sc_degree(dst, zn)
    q1, dinv = _tc1(degp, x, W1)
    accp1 = _sc_agg4(q1.reshape(H * N), src, dst, zhn)
    q2 = _tc2(accp1.reshape(NTILES, H, N), q1, dinv,
              b1.reshape(H, 1), W2)
    accp2 = _sc_agg1(q2.reshape(N), src, dst, zn)
    out = _tc3(accp2, q2, dinv, b2.reshape(1, 1))
    return out.reshape(N, 1)
